# baseline (speedup 1.0000x reference)
import functools

import jax
import jax.numpy as jnp
from jax import lax
from jax.experimental import pallas as pl
from jax.experimental.pallas import tpu as pltpu
from jax.experimental.pallas import tpu_sc as plsc


@functools.lru_cache(maxsize=None)
def _make_gather(V, B):
    info = plsc.get_sparse_core_info()
    NC, NS = info.num_cores, info.num_subcores
    NW = NC * NS
    K = 2
    CHUNK = K * 128
    rows_per_w = B // 128 // NW
    n_chunks = rows_per_w // K
    assert n_chunks % 2 == 0 and n_chunks >= 4
    mesh = plsc.VectorSubcoreMesh(core_axis_name="c", subcore_axis_name="s")

    @functools.partial(
        pl.kernel,
        mesh=mesh,
        out_type=jax.ShapeDtypeStruct((B, 128), jnp.float32),
        scratch_types=[
            pltpu.VMEM((2, K, 128), jnp.int32),
            pltpu.VMEM((2, CHUNK, 128), jnp.float32),
            pltpu.SemaphoreType.DMA,
            pltpu.SemaphoreType.DMA,
        ],
    )
    def gather(table_hbm, idx_hbm, out_hbm, idx_v, rows_v, gsem, ssem):
        wid = lax.axis_index("s") * NC + lax.axis_index("c")
        row0 = wid * rows_per_w

        def fire_gathers(c, b):
            pltpu.sync_copy(idx_hbm.at[pl.ds(row0 + c * K, K)], idx_v.at[b])
            for j in range(K):
                pltpu.async_copy(
                    table_hbm.at[idx_v.at[b].at[j]],
                    rows_v.at[b].at[pl.ds(j * 128, 128)],
                    gsem,
                )

        def wait_gathers(b):
            pltpu.make_async_copy(
                table_hbm.at[pl.ds(0, CHUNK)], rows_v.at[b], gsem
            ).wait()

        def fire_store(c, b):
            pltpu.async_copy(
                rows_v.at[b],
                out_hbm.at[pl.ds((row0 + c * K) * 128, CHUNK)],
                ssem,
            )

        def wait_store(b):
            pltpu.make_async_copy(
                rows_v.at[b], out_hbm.at[pl.ds(0, CHUNK)], ssem
            ).wait()

        fire_gathers(0, 0)
        fire_gathers(1, 1)
        wait_gathers(0)
        fire_store(0, 0)
        wait_gathers(1)
        fire_store(1, 1)

        def body(g, carry):
            c0 = 2 + 2 * g
            wait_store(0)
            fire_gathers(c0, 0)
            wait_store(1)
            fire_gathers(c0 + 1, 1)
            wait_gathers(0)
            fire_store(c0, 0)
            wait_gathers(1)
            fire_store(c0 + 1, 1)
            return carry

        lax.fori_loop(0, (n_chunks - 2) // 2, body, 0)
        wait_store(0)
        wait_store(1)

    return gather


def _depad_body(i_ref, o_ref):
    o_ref[...] = i_ref[:, : o_ref.shape[1]]


@functools.lru_cache(maxsize=None)
def _make_depad(B, D):
    BK = 4096
    return pl.pallas_call(
        _depad_body,
        grid=(B // BK,),
        in_specs=[pl.BlockSpec((BK, 128), lambda i: (i, 0))],
        out_specs=pl.BlockSpec((BK, D), lambda i: (i, 0)),
        out_shape=jax.ShapeDtypeStruct((B, D), jnp.float32),
    )


def kernel(x, w):
    B0, H = x.shape
    V, D = w.shape
    B = B0 * H
    idx2d = x.reshape(B // 128, 128)
    wp = jnp.pad(w, ((0, 0), (0, 128 - D)))
    g = _make_gather(V, B)(wp, idx2d)
    out = _make_depad(B, D)(g)
    return jax.lax.stop_gradient(out.reshape(B0, H, D))
